# baseline (speedup 1.0000x reference)
import functools

import jax
import jax.numpy as jnp
from jax import lax
from jax.experimental import pallas as pl
from jax.experimental.pallas import tpu as pltpu
from jax.experimental.pallas import tpu_sc as plsc

_C = 64
_NX, _NY, _NB = 432, 496, 2
_P = 30000
_PB = _P // _NB
_PBP = 15360
_CHUNK = 1024
_NCH = _PBP // _CHUNK
_GPC = _CHUNK // 16
_ROWS_W = 32
_NPOSMAX = _ROWS_W * _NY
_BLK = 8 * _NY
_RCAP = 256


def _scatter_one(feat_hbm, y_hbm, x_hbm, out_hbm,
                 yv, xv, win, pos_l, w_l, rows, slab, sem,
                 cid, sid, lane, x0, ncell, nblk):
    base_rel = x0 * _NY

    def _init(i, _):
        win[pl.ds(i * 16, 16)] = jnp.full((16,), -1, jnp.int32)
        return 0
    lax.fori_loop(0, _NPOSMAX // 16, _init, 0)

    def _chunk1(ch, _):
        pltpu.sync_copy(y_hbm.at[pl.ds(cid * _PBP + ch * _CHUNK, _CHUNK)], yv)
        pltpu.sync_copy(x_hbm.at[pl.ds(cid * _PBP + ch * _CHUNK, _CHUNK)], xv)

        def _grp(g, _):
            yg = yv[pl.ds(g * 16, 16)]
            xg = xv[pl.ds(g * 16, 16)]
            rel = xg * _NY + yg - base_rel
            m = (rel >= 0) & (rel < ncell)
            pg = cid * _PB + ch * _CHUNK + g * 16 + lane
            _, lastm = plsc.scan_count(rel, mask=m)
            plsc.store_scatter(win, [rel], pg, mask=m & lastm)
            return 0
        lax.fori_loop(0, _GPC, _grp, 0)
        return 0
    lax.fori_loop(0, _NCH, _chunk1, 0)

    def _gather_chunk(ck, n):
        cnt = jnp.minimum(n - ck * _RCAP, _RCAP)
        ng = (cnt + 15) // 16

        def _fire(j, _):
            wv = plsc.load_gather(w_l, [ck * _RCAP + j * 16 + lane])
            pltpu.async_copy(feat_hbm.at[wv], rows.at[pl.ds(j * 16, 16), :], sem)
            return 0
        lax.fori_loop(0, ng, _fire, 0)

        def _drain(j, _):
            z16 = jnp.zeros((16,), jnp.int32)
            pltpu.make_async_copy(feat_hbm.at[z16],
                                  rows.at[pl.ds(j * 16, 16), :], sem).wait()
            return 0
        lax.fori_loop(0, ng, _drain, 0)

    def _scatter_chunk(ck, n, cg):
        cnt = jnp.minimum(n - ck * _RCAP, _RCAP)
        ng = (cnt + 15) // 16

        def _tr(j, _):
            rid = ck * _RCAP + j * 16 + lane
            gm = rid < n
            pos = plsc.load_gather(pos_l, [rid])
            xr = pos // _NY
            yr = pos - xr * _NY
            rl = j * 16 + lane
            for c in range(16):
                vals = plsc.load_gather(rows, [rl, jnp.full((16,), cg * 16 + c,
                                                            jnp.int32)])
                plsc.store_scatter(slab, [jnp.full((16,), c, jnp.int32), xr, yr],
                                   vals, mask=gm)
            return 0
        lax.fori_loop(0, ng, _tr, 0)

    def _blk2(blk, _):
        def _compact(k, n):
            w = win[pl.ds(blk * _BLK + k * 16, 16)]
            m = w >= 0
            mi = m.astype(jnp.int32)
            il = jnp.full((16,), n, jnp.int32) + plsc.cumsum(mi) - 1
            plsc.store_scatter(pos_l, [il], k * 16 + lane, mask=m)
            plsc.store_scatter(w_l, [il], w, mask=m)
            return n + jnp.sum(mi)
        n = lax.fori_loop(0, _BLK // 16, _compact, jnp.int32(0))

        nck = (n + _RCAP - 1) // _RCAP

        @pl.when(n > 0)
        def _():
            _gather_chunk(jnp.int32(0), n)

        for cg in range(_C // 16):
            def _ck_body(ck, _, cg=cg):
                if cg == 0:
                    do_g = ck > 0
                else:
                    do_g = (ck > 0) | (nck > 1)

                @pl.when(do_g)
                def _():
                    _gather_chunk(ck, n)
                _scatter_chunk(ck, n, cg)
                return 0
            lax.fori_loop(0, nck, functools.partial(_ck_body, cg=cg), 0)

            pltpu.sync_copy(
                slab, out_hbm.at[cid, pl.ds(cg * 16, 16),
                                 pl.ds(x0 + blk * 8, 8), :])

            def _undo(j, _):
                gm = j * 16 + lane < n
                pos = plsc.load_gather(pos_l, [j * 16 + lane])
                xr = pos // _NY
                yr = pos - xr * _NY
                zz = jnp.zeros((16,), jnp.float32)
                for c in range(16):
                    plsc.store_scatter(slab,
                                       [jnp.full((16,), c, jnp.int32), xr, yr],
                                       zz, mask=gm)
                return 0
            lax.fori_loop(0, (n + 15) // 16, _undo, 0)
        return 0
    lax.fori_loop(0, nblk, _blk2, 0)


def _body(feat0, y0h, x0h, feat1, y1h, x1h, out0, out1,
          yv, xv, win, pos_l, w_l, rows, slab, sem):
    cid = lax.axis_index("c")
    sid = lax.axis_index("s")
    lane = jnp.arange(16, dtype=jnp.int32)

    x0 = sid * 32 - jnp.maximum(sid - 6, 0) * 8
    ncol = jnp.where(sid < 6, 32, 24)
    ncell = ncol * _NY
    nblk = ncol // 8

    def _z1(i, _):
        ch = i // (8 * _NY // 16)
        r = (i % (8 * _NY // 16)) // (_NY // 16)
        yk = i % (_NY // 16)
        slab[ch, r, pl.ds(yk * 16, 16)] = jnp.zeros((16,), jnp.float32)
        return 0
    lax.fori_loop(0, 16 * 8 * (_NY // 16), _z1, 0)

    def _zp(k, _):
        w_l[pl.ds(k * 16, 16)] = jnp.zeros((16,), jnp.int32)
        return 0
    lax.fori_loop(0, _BLK // 16, _zp, 0)

    args = (yv, xv, win, pos_l, w_l, rows, slab, sem,
            cid, sid, lane, x0, ncell, nblk)
    _scatter_one(feat0, y0h, x0h, out0, *args)
    _scatter_one(feat1, y1h, x1h, out1, *args)


@jax.jit
def kernel(pillar_features, voxel_coords, ppillar_features, pvoxel_coords):
    def _prep(coords):
        c = coords.astype(jnp.int32)
        y = c[:, 2].reshape(_NB, _PB)
        x = c[:, 3].reshape(_NB, _PB)
        y = jnp.pad(y, ((0, 0), (0, _PBP - _PB)), constant_values=_NY)
        x = jnp.pad(x, ((0, 0), (0, _PBP - _PB)), constant_values=_NX)
        return y.reshape(-1), x.reshape(-1)

    y0h, x0h = _prep(voxel_coords)
    y1h, x1h = _prep(pvoxel_coords)
    feat0 = jnp.pad(pillar_features, ((0, 0), (0, _C)))
    feat1 = jnp.pad(ppillar_features, ((0, 0), (0, _C)))

    run = pl.kernel(
        _body,
        out_type=(
            jax.ShapeDtypeStruct((_NB, _C, _NX, _NY), jnp.float32),
            jax.ShapeDtypeStruct((_NB, _C, _NX, _NY), jnp.float32),
        ),
        mesh=plsc.VectorSubcoreMesh(core_axis_name="c", subcore_axis_name="s"),
        compiler_params=pltpu.CompilerParams(needs_layout_passes=False,
                                             use_tc_tiling_on_sc=True),
        scratch_types=(
            pltpu.VMEM((_CHUNK,), jnp.int32),
            pltpu.VMEM((_CHUNK,), jnp.int32),
            pltpu.VMEM((_NPOSMAX,), jnp.int32),
            pltpu.VMEM((_BLK,), jnp.int32),
            pltpu.VMEM((_BLK,), jnp.int32),
            pltpu.VMEM((_RCAP, 2 * _C), jnp.float32),
            pltpu.VMEM((16, 8, _NY), jnp.float32),
            pltpu.SemaphoreType.DMA,
        ),
    )
    o0, o1 = run(feat0, y0h, x0h, feat1, y1h, x1h)
    return jnp.swapaxes(o0, 2, 3), jnp.swapaxes(o1, 2, 3)


# speedup vs baseline: 1.3084x; 1.3084x over previous
import functools

import jax
import jax.numpy as jnp
from jax import lax
from jax.experimental import pallas as pl
from jax.experimental.pallas import tpu as pltpu
from jax.experimental.pallas import tpu_sc as plsc

_C = 64
_NX, _NY, _NB = 432, 496, 2
_P = 30000
_PB = _P // _NB
_PBP = 15360
_CHUNK = 1024
_NCH = _PBP // _CHUNK
_GPC = _CHUNK // 16
_ROWS_W = 32
_NPOSMAX = _ROWS_W * _NY
_BLK = 8 * _NY
_RCAP = 288


def _scatter_one(feat_hbm, y_hbm, x_hbm, out_hbm,
                 yv, xv, win, pos_l, w_l, rows, slab, sem, ysem, xsem,
                 cid, sid, lane, x0, ncell, nblk):
    base_rel = x0 * _NY

    def _init(i, _):
        win[pl.ds(i * 16, 16)] = jnp.full((16,), -1, jnp.int32)
        return 0
    lax.fori_loop(0, _NPOSMAX // 16, _init, 0)

    def _stage(ch, par):
        base = cid * _PBP + ch * _CHUNK
        return (pltpu.async_copy(y_hbm.at[pl.ds(base, _CHUNK)], yv.at[par], ysem),
                pltpu.async_copy(x_hbm.at[pl.ds(base, _CHUNK)], xv.at[par], xsem))

    descs = _stage(0, 0)
    for ch in range(_NCH):
        par = ch % 2
        descs[0].wait()
        descs[1].wait()
        if ch + 1 < _NCH:
            descs = _stage(ch + 1, (ch + 1) % 2)

        def _grp(g, _, ch=ch, par=par):
            yg = yv[par, pl.ds(g * 16, 16)]
            xg = xv[par, pl.ds(g * 16, 16)]
            rel = xg * _NY + yg - base_rel
            m = (rel >= 0) & (rel < ncell)
            pg = cid * _PB + ch * _CHUNK + g * 16 + lane
            _, lastm = plsc.scan_count(rel, mask=m)
            plsc.store_scatter(win, [rel], pg, mask=m & lastm)
            return 0
        lax.fori_loop(0, _GPC, _grp, 0)

    def _gather_chunk(ck, n):
        cnt = jnp.minimum(n - ck * _RCAP, _RCAP)
        ng = (cnt + 15) // 16

        def _fire(j, _):
            wv = plsc.load_gather(w_l, [ck * _RCAP + j * 16 + lane])
            pltpu.async_copy(feat_hbm.at[wv], rows.at[pl.ds(j * 16, 16), :], sem)
            return 0
        lax.fori_loop(0, ng, _fire, 0)

        def _drain(j, _):
            z16 = jnp.zeros((16,), jnp.int32)
            pltpu.make_async_copy(feat_hbm.at[z16],
                                  rows.at[pl.ds(j * 16, 16), :], sem).wait()
            return 0
        lax.fori_loop(0, ng, _drain, 0)

    def _scatter_chunk(ck, n, cg):
        cnt = jnp.minimum(n - ck * _RCAP, _RCAP)
        ng = (cnt + 15) // 16

        def _tr(j, _):
            rid = ck * _RCAP + j * 16 + lane
            gm = rid < n
            pos = plsc.load_gather(pos_l, [rid])
            xr = pos // _NY
            yr = pos - xr * _NY
            rl = j * 16 + lane
            for c in range(16):
                vals = plsc.load_gather(rows, [rl, jnp.full((16,), cg * 16 + c,
                                                            jnp.int32)])
                plsc.store_scatter(slab, [jnp.full((16,), c, jnp.int32), xr, yr],
                                   vals, mask=gm)
            return 0
        lax.fori_loop(0, ng, _tr, 0)

    def _blk2(blk, _):
        def _compact(k, n):
            w = win[pl.ds(blk * _BLK + k * 16, 16)]
            m = w >= 0
            mi = m.astype(jnp.int32)
            il = jnp.full((16,), n, jnp.int32) + plsc.cumsum(mi) - 1
            plsc.store_scatter(pos_l, [il], k * 16 + lane, mask=m)
            plsc.store_scatter(w_l, [il], w, mask=m)
            return n + jnp.sum(mi)
        n = lax.fori_loop(0, _BLK // 16, _compact, jnp.int32(0))

        nck = (n + _RCAP - 1) // _RCAP

        @pl.when(n > 0)
        def _():
            _gather_chunk(jnp.int32(0), n)

        for cg in range(_C // 16):
            def _ck_body(ck, _, cg=cg):
                if cg == 0:
                    do_g = ck > 0
                else:
                    do_g = (ck > 0) | (nck > 1)

                @pl.when(do_g)
                def _():
                    _gather_chunk(ck, n)
                _scatter_chunk(ck, n, cg)
                return 0
            lax.fori_loop(0, nck, functools.partial(_ck_body, cg=cg), 0)

            pltpu.sync_copy(
                slab, out_hbm.at[cid, pl.ds(cg * 16, 16),
                                 pl.ds(x0 + blk * 8, 8), :])

            def _undo(j, _):
                gm = j * 16 + lane < n
                pos = plsc.load_gather(pos_l, [j * 16 + lane])
                xr = pos // _NY
                yr = pos - xr * _NY
                zz = jnp.zeros((16,), jnp.float32)
                for c in range(16):
                    plsc.store_scatter(slab,
                                       [jnp.full((16,), c, jnp.int32), xr, yr],
                                       zz, mask=gm)
                return 0
            lax.fori_loop(0, (n + 15) // 16, _undo, 0)
        return 0
    lax.fori_loop(0, nblk, _blk2, 0)


def _body(feat0, y0h, x0h, feat1, y1h, x1h, out0, out1,
          yv, xv, win, pos_l, w_l, rows, slab, sem, ysem, xsem):
    cid = lax.axis_index("c")
    sid = lax.axis_index("s")
    lane = jnp.arange(16, dtype=jnp.int32)

    x0 = sid * 32 - jnp.maximum(sid - 6, 0) * 8
    ncol = jnp.where(sid < 6, 32, 24)
    ncell = ncol * _NY
    nblk = ncol // 8

    def _z1(i, _):
        ch = i // (8 * _NY // 16)
        r = (i % (8 * _NY // 16)) // (_NY // 16)
        yk = i % (_NY // 16)
        slab[ch, r, pl.ds(yk * 16, 16)] = jnp.zeros((16,), jnp.float32)
        return 0
    lax.fori_loop(0, 16 * 8 * (_NY // 16), _z1, 0)

    def _zp(k, _):
        w_l[pl.ds(k * 16, 16)] = jnp.zeros((16,), jnp.int32)
        return 0
    lax.fori_loop(0, _BLK // 16, _zp, 0)

    args = (yv, xv, win, pos_l, w_l, rows, slab, sem, ysem, xsem,
            cid, sid, lane, x0, ncell, nblk)
    _scatter_one(feat0, y0h, x0h, out0, *args)
    _scatter_one(feat1, y1h, x1h, out1, *args)


@jax.jit
def kernel(pillar_features, voxel_coords, ppillar_features, pvoxel_coords):
    def _prep(coords):
        c = coords.astype(jnp.int32)
        y = c[:, 2].reshape(_NB, _PB)
        x = c[:, 3].reshape(_NB, _PB)
        y = jnp.pad(y, ((0, 0), (0, _PBP - _PB)), constant_values=_NY)
        x = jnp.pad(x, ((0, 0), (0, _PBP - _PB)), constant_values=_NX)
        return y.reshape(-1), x.reshape(-1)

    y0h, x0h = _prep(voxel_coords)
    y1h, x1h = _prep(pvoxel_coords)
    feat0 = jnp.pad(pillar_features, ((0, 0), (0, _C)))
    feat1 = jnp.pad(ppillar_features, ((0, 0), (0, _C)))

    run = pl.kernel(
        _body,
        out_type=(
            jax.ShapeDtypeStruct((_NB, _C, _NX, _NY), jnp.float32),
            jax.ShapeDtypeStruct((_NB, _C, _NX, _NY), jnp.float32),
        ),
        mesh=plsc.VectorSubcoreMesh(core_axis_name="c", subcore_axis_name="s"),
        compiler_params=pltpu.CompilerParams(needs_layout_passes=False,
                                             use_tc_tiling_on_sc=True),
        scratch_types=(
            pltpu.VMEM((2, _CHUNK), jnp.int32),
            pltpu.VMEM((2, _CHUNK), jnp.int32),
            pltpu.VMEM((_NPOSMAX,), jnp.int32),
            pltpu.VMEM((_BLK,), jnp.int32),
            pltpu.VMEM((_BLK,), jnp.int32),
            pltpu.VMEM((_RCAP, 2 * _C), jnp.float32),
            pltpu.VMEM((16, 8, _NY), jnp.float32),
            pltpu.SemaphoreType.DMA,
            pltpu.SemaphoreType.DMA,
            pltpu.SemaphoreType.DMA,
        ),
    )
    o0, o1 = run(feat0, y0h, x0h, feat1, y1h, x1h)
    return jnp.swapaxes(o0, 2, 3), jnp.swapaxes(o1, 2, 3)
